# baseline (speedup 1.0000x reference)
import functools

import jax
import jax.numpy as jnp
import numpy as np
from jax import lax
from jax.experimental import pallas as pl
from jax.experimental.pallas import tpu as pltpu
from jax.experimental.pallas import tpu_sc as plsc

B = 1024
K = 1000
EMB = 64
C0 = 512
C1 = K - C0

_B255 = jnp.int32(0xFF)


def _fp8_e4m3_table():
    t = np.zeros(256, np.float32)
    for b in range(256):
        s = -1.0 if (b >> 7) & 1 else 1.0
        e = (b >> 3) & 0xF
        m = b & 7
        if (b & 0x7F) == 0x7F:
            t[b] = 0.0
        elif e == 0:
            t[b] = s * (m / 8.0) * 2.0 ** -6
        else:
            t[b] = s * (1 + m / 8.0) * 2.0 ** (e - 7)
    return t


_TABLE = _fp8_e4m3_table()


def _sc_kernel_make():
    info = plsc.get_sparse_core_info()
    nc, ns = info.num_cores, info.num_subcores
    nw = nc * ns
    bw = B // nw

    mesh = plsc.VectorSubcoreMesh(core_axis_name="c", subcore_axis_name="s")

    @functools.partial(
        pl.kernel,
        mesh=mesh,
        compiler_params=pltpu.CompilerParams(
            needs_layout_passes=False, use_tc_tiling_on_sc=False),
        out_type=[
            jax.ShapeDtypeStruct((B,), jnp.float32),
            jax.ShapeDtypeStruct((B,), jnp.float32),
        ],
        scratch_types=[
            pltpu.VMEM((256,), jnp.float32),
            pltpu.VMEM((bw,), jnp.int32),
            pltpu.VMEM((bw,), jnp.int32),
            pltpu.VMEM((bw * K,), jnp.int32),
            pltpu.VMEM((bw, EMB), jnp.uint8),
            pltpu.VMEM((bw, EMB), jnp.uint8),
            pltpu.VMEM((bw, EMB), jnp.float32),
            pltpu.VMEM((bw, EMB), jnp.float32),
            pltpu.VMEM((C0, EMB), jnp.uint8),
            pltpu.VMEM((C0, EMB), jnp.uint8),
            pltpu.VMEM((16, 16), jnp.float32),
            pltpu.VMEM((bw, 16), jnp.float32),
            pltpu.VMEM((bw,), jnp.float32),
            pltpu.VMEM((bw,), jnp.float32),
            pltpu.SemaphoreType.DMA,
            pltpu.SemaphoreType.DMA,
            pltpu.SemaphoreType.DMA,
        ],
    )
    def sc_kernel(tab_hbm, cidx_hbm, tidx_hbm, av_hbm, v8_hbm, u8_hbm,
                  scores_hbm, denom_hbm,
                  tab_v, cidx_v, tidx_v, av_v, crows8_v, trows8_v,
                  crows_v, trows_v,
                  rbuf0, rbuf1, qbuf, accbuf, sc_v, dn_v,
                  sem_s, sem0, sem1):
        wid = lax.axis_index("s") * nc + lax.axis_index("c")
        base_b = wid * bw
        lanes = lax.iota(jnp.int32, 16)

        def col(l):
            return jnp.full((16,), l, jnp.int32)

        cp_b = pltpu.make_async_copy(tab_hbm, tab_v, sem_s)
        cp_c = pltpu.make_async_copy(cidx_hbm.at[pl.ds(base_b, bw)], cidx_v, sem_s)
        cp_t = pltpu.make_async_copy(tidx_hbm.at[pl.ds(base_b, bw)], tidx_v, sem_s)
        cp_a = pltpu.make_async_copy(av_hbm.at[pl.ds(base_b * K, bw * K)], av_v, sem_s)
        cp_b.start(); cp_c.start(); cp_t.start(); cp_a.start()
        cp_b.wait(); cp_c.wait(); cp_t.wait(); cp_a.wait()
        cp_cr = pltpu.make_async_copy(v8_hbm.at[cidx_v], crows8_v, sem_s)
        cp_tr = pltpu.make_async_copy(u8_hbm.at[tidx_v], trows8_v, sem_s)
        cp_cr.start(); cp_tr.start()

        rbufs = (rbuf0, rbuf1)
        sems = (sem0, sem1)

        def start_gather(lb, t, buf, sem):
            if t == 0:
                src = u8_hbm.at[av_v.at[pl.ds(lb * K, C0)]]
                pltpu.make_async_copy(src, buf, sem).start()
            else:
                src = u8_hbm.at[av_v.at[pl.ds(lb * K + C0, C1)]]
                pltpu.make_async_copy(src, buf.at[pl.ds(0, C1)], sem).start()

        def wait_gather(t, buf, sem):
            if t == 0:
                src = u8_hbm.at[av_v.at[pl.ds(0, C0)]]
                pltpu.make_async_copy(src, buf, sem).wait()
            else:
                src = u8_hbm.at[av_v.at[pl.ds(C0, C1)]]
                pltpu.make_async_copy(src, buf.at[pl.ds(0, C1)], sem).wait()

        start_gather(0, 0, rbuf0, sem0)
        start_gather(0, 1, rbuf1, sem1)
        cp_cr.wait(); cp_tr.wait()

        def decode_fp8(w):
            f0 = plsc.load_gather(tab_v, [w & _B255])
            f1 = plsc.load_gather(tab_v, [(w >> 8) & _B255])
            f2 = plsc.load_gather(tab_v, [(w >> 16) & _B255])
            f3 = plsc.load_gather(tab_v, [(w >> 24) & _B255])
            return f0, f1, f2, f3

        for lb in range(bw):
            w = plsc.bitcast(crows8_v[lb], jnp.int32)
            f0, f1, f2, f3 = decode_fp8(w)
            crows_v[lb, pl.ds(0, 16)] = f0
            crows_v[lb, pl.ds(16, 16)] = f1
            crows_v[lb, pl.ds(32, 16)] = f2
            crows_v[lb, pl.ds(48, 16)] = f3
            w = plsc.bitcast(trows8_v[lb], jnp.int32)
            f0, f1, f2, f3 = decode_fp8(w)
            trows_v[lb, pl.ds(0, 16)] = f0
            trows_v[lb, pl.ds(16, 16)] = f1
            trows_v[lb, pl.ds(32, 16)] = f2
            trows_v[lb, pl.ds(48, 16)] = f3

        def compute_chunk(lb, t, rbuf, acc):
            cc0 = crows_v[lb, pl.ds(0, 16)]
            cc1 = crows_v[lb, pl.ds(16, 16)]
            cc2 = crows_v[lb, pl.ds(32, 16)]
            cc3 = crows_v[lb, pl.ds(48, 16)]

            def group(gi, acc):
                for r in range(16):
                    row = gi * 16 + r
                    w = plsc.bitcast(rbuf[row], jnp.int32)
                    f0, f1, f2, f3 = decode_fp8(w)
                    q = (f0 * cc0 + f1 * cc1) + (f2 * cc2 + f3 * cc3)
                    qbuf[r] = q
                g = [plsc.load_gather(qbuf, [lanes, col(l)]) for l in range(16)]
                while len(g) > 1:
                    g = [a + b for a, b in zip(g[::2], g[1::2])]
                e = jnp.exp(g[0])
                if t == 1:
                    e = jnp.where(gi * 16 + lanes < C1, e, jnp.float32(0.0))
                return acc + e

            ngroups = C0 // 16 if t == 0 else (C1 + 15) // 16
            return lax.fori_loop(0, ngroups, group, acc)

        def body(i, acc):
            lb = i
            for t in range(2):
                wait_gather(t, rbufs[t], sems[t])
                acc = compute_chunk(lb, t, rbufs[t], acc)

                @pl.when(lb + 1 < bw)
                def _():
                    start_gather(lb + 1, t, rbufs[t], sems[t])
            accbuf[lb] = acc
            return jnp.zeros((16,), jnp.float32)

        lax.fori_loop(0, bw, body, jnp.zeros((16,), jnp.float32))

        for half in range(bw // 16):
            base = half * 16
            g = [plsc.load_gather(accbuf, [base + lanes, col(l)]) for l in range(16)]
            while len(g) > 1:
                g = [a + b for a, b in zip(g[::2], g[1::2])]
            dn_v[pl.ds(base, 16)] = g[0]

        for half in range(bw // 16):
            for r in range(16):
                lb = half * 16 + r
                q = crows_v[lb, pl.ds(0, 16)] * trows_v[lb, pl.ds(0, 16)]
                q = q + crows_v[lb, pl.ds(16, 16)] * trows_v[lb, pl.ds(16, 16)]
                q = q + crows_v[lb, pl.ds(32, 16)] * trows_v[lb, pl.ds(32, 16)]
                q = q + crows_v[lb, pl.ds(48, 16)] * trows_v[lb, pl.ds(48, 16)]
                qbuf[r] = q
            g = [plsc.load_gather(qbuf, [lanes, col(l)]) for l in range(16)]
            while len(g) > 1:
                g = [a + b for a, b in zip(g[::2], g[1::2])]
            sc_v[pl.ds(half * 16, 16)] = g[0]

        pltpu.sync_copy(sc_v, scores_hbm.at[pl.ds(base_b, bw)])
        pltpu.sync_copy(dn_v, denom_hbm.at[pl.ds(base_b, bw)])

    return sc_kernel


_sc_kernel = _sc_kernel_make()


def _finish(s_ref, d_ref, o_ref):
    nll = -jnp.mean(s_ref[...] - jnp.log(d_ref[...]))
    o_ref[...] = jnp.full((8, 128), nll, jnp.float32)


_finish_call = pl.pallas_call(
    _finish,
    out_shape=jax.ShapeDtypeStruct((8, 128), jnp.float32),
)


def _fp8_bits(x):
    return lax.bitcast_convert_type(x.astype(jnp.float8_e4m3fn), jnp.uint8)


@jax.jit
def kernel(center_words, target_words, all_vocabs, V, U):
    cidx = center_words.reshape(-1).astype(jnp.int32)
    tidx = target_words.reshape(-1).astype(jnp.int32)
    av = all_vocabs.astype(jnp.int32).reshape(-1)
    tab = jnp.asarray(_TABLE)
    scores, denom = _sc_kernel(tab, cidx, tidx, av, _fp8_bits(V), _fp8_bits(U))
    out = _finish_call(scores.reshape(8, 128), denom.reshape(8, 128))
    return out[0, 0]


# speedup vs baseline: 1.3087x; 1.3087x over previous
import functools

import jax
import jax.numpy as jnp
import numpy as np
from jax import lax
from jax.experimental import pallas as pl
from jax.experimental.pallas import tpu as pltpu
from jax.experimental.pallas import tpu_sc as plsc

B = 1024
K = 1000
EMB = 64
C0 = 512
C1 = K - C0


def _fp8_e4m3_table():
    t = np.zeros(256, np.float32)
    for b in range(256):
        s = -1.0 if (b >> 7) & 1 else 1.0
        e = (b >> 3) & 0xF
        m = b & 7
        if (b & 0x7F) == 0x7F:
            t[b] = 0.0
        elif e == 0:
            t[b] = s * (m / 8.0) * 2.0 ** -6
        else:
            t[b] = s * (1 + m / 8.0) * 2.0 ** (e - 7)
    return t


_TABLE = _fp8_e4m3_table()


def _sc_kernel_make():
    info = plsc.get_sparse_core_info()
    nc, ns = info.num_cores, info.num_subcores
    nw = nc * ns
    bw = B // nw

    mesh = plsc.VectorSubcoreMesh(core_axis_name="c", subcore_axis_name="s")

    @functools.partial(
        pl.kernel,
        mesh=mesh,
        compiler_params=pltpu.CompilerParams(
            needs_layout_passes=False, use_tc_tiling_on_sc=False),
        out_type=[
            jax.ShapeDtypeStruct((B,), jnp.float32),
            jax.ShapeDtypeStruct((B,), jnp.float32),
        ],
        scratch_types=[
            pltpu.VMEM((256,), jnp.float32),
            pltpu.VMEM((bw,), jnp.int32),
            pltpu.VMEM((bw,), jnp.int32),
            pltpu.VMEM((bw * K,), jnp.int32),
            pltpu.VMEM((bw, EMB), jnp.uint8),
            pltpu.VMEM((bw, EMB), jnp.uint8),
            pltpu.VMEM((bw, EMB), jnp.float32),
            pltpu.VMEM((bw, EMB), jnp.float32),
            pltpu.VMEM((bw, EMB // 2), jnp.bfloat16),
            pltpu.VMEM((bw, EMB // 2), jnp.bfloat16),
            pltpu.VMEM((C0, EMB), jnp.uint8),
            pltpu.VMEM((C0, EMB), jnp.uint8),
            pltpu.VMEM((16, 16), jnp.float32),
            pltpu.VMEM((bw, 16), jnp.float32),
            pltpu.VMEM((bw,), jnp.float32),
            pltpu.VMEM((bw,), jnp.float32),
            pltpu.SemaphoreType.DMA,
            pltpu.SemaphoreType.DMA,
            pltpu.SemaphoreType.DMA,
        ],
    )
    def sc_kernel(tab_hbm, cidx_hbm, tidx_hbm, av_hbm, v8_hbm, u8_hbm,
                  scores_hbm, denom_hbm,
                  tab_v, cidx_v, tidx_v, av_v, crows8_v, trows8_v,
                  crows_v, trows_v, clo_v, chi_v,
                  rbuf0, rbuf1, qbuf, accbuf, sc_v, dn_v,
                  sem_s, sem0, sem1):
        wid = lax.axis_index("s") * nc + lax.axis_index("c")
        base_b = wid * bw
        lanes = lax.iota(jnp.int32, 16)

        def col(l):
            return jnp.full((16,), l, jnp.int32)

        _B255 = jnp.full((16,), 0xFF, jnp.int32)
        _MPAY = jnp.full((16,), 0x07F007F0, jnp.uint32)
        _MREB = jnp.full((16,), 0x3C003C00, jnp.uint32)
        _MSLO = jnp.full((16,), 0x00800080, jnp.uint32)
        _MSHI = jnp.full((16,), 0x80008000, jnp.uint32)
        _HI32 = jnp.full((16,), 0xFFFF0000, jnp.uint32)

        cp_b = pltpu.make_async_copy(tab_hbm, tab_v, sem_s)
        cp_c = pltpu.make_async_copy(cidx_hbm.at[pl.ds(base_b, bw)], cidx_v, sem_s)
        cp_t = pltpu.make_async_copy(tidx_hbm.at[pl.ds(base_b, bw)], tidx_v, sem_s)
        cp_a = pltpu.make_async_copy(av_hbm.at[pl.ds(base_b * K, bw * K)], av_v, sem_s)
        cp_b.start(); cp_c.start(); cp_t.start(); cp_a.start()
        cp_b.wait(); cp_c.wait(); cp_t.wait(); cp_a.wait()
        cp_cr = pltpu.make_async_copy(v8_hbm.at[cidx_v], crows8_v, sem_s)
        cp_tr = pltpu.make_async_copy(u8_hbm.at[tidx_v], trows8_v, sem_s)
        cp_cr.start(); cp_tr.start()

        rbufs = (rbuf0, rbuf1)
        sems = (sem0, sem1)

        def start_gather(lb, t, buf, sem):
            if t == 0:
                src = u8_hbm.at[av_v.at[pl.ds(lb * K, C0)]]
                pltpu.make_async_copy(src, buf, sem).start()
            else:
                src = u8_hbm.at[av_v.at[pl.ds(lb * K + C0, C1)]]
                pltpu.make_async_copy(src, buf.at[pl.ds(0, C1)], sem).start()

        def wait_gather(t, buf, sem):
            if t == 0:
                src = u8_hbm.at[av_v.at[pl.ds(0, C0)]]
                pltpu.make_async_copy(src, buf, sem).wait()
            else:
                src = u8_hbm.at[av_v.at[pl.ds(C0, C1)]]
                pltpu.make_async_copy(src, buf.at[pl.ds(0, C1)], sem).wait()

        start_gather(0, 0, rbuf0, sem0)
        start_gather(0, 1, rbuf1, sem1)
        cp_cr.wait(); cp_tr.wait()

        def decode_fp8(w):
            f0 = plsc.load_gather(tab_v, [w & _B255])
            f1 = plsc.load_gather(tab_v, [(w >> 8) & _B255])
            f2 = plsc.load_gather(tab_v, [(w >> 16) & _B255])
            f3 = plsc.load_gather(tab_v, [(w >> 24) & _B255])
            return f0, f1, f2, f3

        for lb in range(bw):
            w = plsc.bitcast(crows8_v[lb], jnp.int32)
            f0, f1, f2, f3 = decode_fp8(w)
            crows_v[lb, pl.ds(0, 16)] = f0
            crows_v[lb, pl.ds(16, 16)] = f1
            crows_v[lb, pl.ds(32, 16)] = f2
            crows_v[lb, pl.ds(48, 16)] = f3
            clo_v[lb] = plsc.pack(f0, f2, format=plsc.PackFormat.INTERLEAVED)
            chi_v[lb] = plsc.pack(f1, f3, format=plsc.PackFormat.INTERLEAVED)
            w = plsc.bitcast(trows8_v[lb], jnp.int32)
            f0, f1, f2, f3 = decode_fp8(w)
            trows_v[lb, pl.ds(0, 16)] = f0
            trows_v[lb, pl.ds(16, 16)] = f1
            trows_v[lb, pl.ds(32, 16)] = f2
            trows_v[lb, pl.ds(48, 16)] = f3

        def compute_chunk(lb, t, rbuf, acc):
            ccl = clo_v[lb]
            cch = chi_v[lb]

            def group(gi, acc):
                for r in range(16):
                    row = gi * 16 + r
                    w = plsc.bitcast(rbuf[row], jnp.uint32)
                    lo = (((w << 4) & _MPAY) + _MREB) | ((w & _MSLO) << 8)
                    hi = (((w >> 4) & _MPAY) + _MREB) | (w & _MSHI)
                    q32 = (plsc.bitcast(lo, jnp.bfloat16) * ccl
                           + plsc.bitcast(hi, jnp.bfloat16) * cch)
                    qw = plsc.bitcast(q32, jnp.uint32)
                    qe = plsc.bitcast(qw << 16, jnp.float32)
                    qo = plsc.bitcast(qw & _HI32, jnp.float32)
                    qbuf[r] = qe + qo
                g = [plsc.load_gather(qbuf, [lanes, col(l)]) for l in range(16)]
                while len(g) > 1:
                    g = [a + b for a, b in zip(g[::2], g[1::2])]
                e = jnp.exp(g[0])
                if t == 1:
                    e = jnp.where(gi * 16 + lanes < C1, e, jnp.float32(0.0))
                return acc + e

            ngroups = C0 // 16 if t == 0 else (C1 + 15) // 16
            return lax.fori_loop(0, ngroups, group, acc)

        def body(i, acc):
            lb = i
            for t in range(2):
                wait_gather(t, rbufs[t], sems[t])
                acc = compute_chunk(lb, t, rbufs[t], acc)

                @pl.when(lb + 1 < bw)
                def _():
                    start_gather(lb + 1, t, rbufs[t], sems[t])
            accbuf[lb] = acc
            return jnp.zeros((16,), jnp.float32)

        lax.fori_loop(0, bw, body, jnp.zeros((16,), jnp.float32))

        for half in range(bw // 16):
            base = half * 16
            g = [plsc.load_gather(accbuf, [base + lanes, col(l)]) for l in range(16)]
            while len(g) > 1:
                g = [a + b for a, b in zip(g[::2], g[1::2])]
            dn_v[pl.ds(base, 16)] = g[0]

        for half in range(bw // 16):
            for r in range(16):
                lb = half * 16 + r
                q = crows_v[lb, pl.ds(0, 16)] * trows_v[lb, pl.ds(0, 16)]
                q = q + crows_v[lb, pl.ds(16, 16)] * trows_v[lb, pl.ds(16, 16)]
                q = q + crows_v[lb, pl.ds(32, 16)] * trows_v[lb, pl.ds(32, 16)]
                q = q + crows_v[lb, pl.ds(48, 16)] * trows_v[lb, pl.ds(48, 16)]
                qbuf[r] = q
            g = [plsc.load_gather(qbuf, [lanes, col(l)]) for l in range(16)]
            while len(g) > 1:
                g = [a + b for a, b in zip(g[::2], g[1::2])]
            sc_v[pl.ds(half * 16, 16)] = g[0]

        pltpu.sync_copy(sc_v, scores_hbm.at[pl.ds(base_b, bw)])
        pltpu.sync_copy(dn_v, denom_hbm.at[pl.ds(base_b, bw)])

    return sc_kernel


_sc_kernel = _sc_kernel_make()


def _finish(s_ref, d_ref, o_ref):
    nll = -jnp.mean(s_ref[...] - jnp.log(d_ref[...]))
    o_ref[...] = jnp.full((8, 128), nll, jnp.float32)


_finish_call = pl.pallas_call(
    _finish,
    out_shape=jax.ShapeDtypeStruct((8, 128), jnp.float32),
)


def _fp8_bits(x):
    return lax.bitcast_convert_type(x.astype(jnp.float8_e4m3fn), jnp.uint8)


@jax.jit
def kernel(center_words, target_words, all_vocabs, V, U):
    cidx = center_words.reshape(-1).astype(jnp.int32)
    tidx = target_words.reshape(-1).astype(jnp.int32)
    av = all_vocabs.astype(jnp.int32).reshape(-1)
    tab = jnp.asarray(_TABLE)
    scores, denom = _sc_kernel(tab, cidx, tidx, av, _fp8_bits(V), _fp8_bits(U))
    out = _finish_call(scores.reshape(8, 128), denom.reshape(8, 128))
    return out[0, 0]


# speedup vs baseline: 1.4446x; 1.1038x over previous
import functools

import jax
import jax.numpy as jnp
from jax import lax
from jax.experimental import pallas as pl
from jax.experimental.pallas import tpu as pltpu
from jax.experimental.pallas import tpu_sc as plsc

B = 1024
K = 1000
EMB = 64
C0 = 512
C1 = K - C0


def _sc_kernel_make():
    info = plsc.get_sparse_core_info()
    nc, ns = info.num_cores, info.num_subcores
    nw = nc * ns
    bw = B // nw

    mesh = plsc.VectorSubcoreMesh(core_axis_name="c", subcore_axis_name="s")

    @functools.partial(
        pl.kernel,
        mesh=mesh,
        compiler_params=pltpu.CompilerParams(
            needs_layout_passes=False, use_tc_tiling_on_sc=False),
        out_type=[
            jax.ShapeDtypeStruct((B,), jnp.float32),
            jax.ShapeDtypeStruct((B,), jnp.float32),
        ],
        scratch_types=[
            pltpu.VMEM((bw,), jnp.int32),
            pltpu.VMEM((bw,), jnp.int32),
            pltpu.VMEM((bw * K,), jnp.int32),
            pltpu.VMEM((bw, EMB), jnp.bfloat16),
            pltpu.VMEM((bw, EMB), jnp.bfloat16),
            pltpu.VMEM((bw, EMB), jnp.float32),
            pltpu.VMEM((bw, EMB), jnp.float32),
            pltpu.VMEM((C0, EMB), jnp.bfloat16),
            pltpu.VMEM((C0, EMB), jnp.bfloat16),
            pltpu.VMEM((16, 16), jnp.float32),
            pltpu.VMEM((bw, 16), jnp.float32),
            pltpu.VMEM((bw,), jnp.float32),
            pltpu.VMEM((bw,), jnp.float32),
            pltpu.SemaphoreType.DMA,
            pltpu.SemaphoreType.DMA,
            pltpu.SemaphoreType.DMA,
        ],
    )
    def sc_kernel(cidx_hbm, tidx_hbm, av_hbm, w16_hbm,
                  scores_hbm, denom_hbm,
                  cidx_v, tidx_v, av_v, crows16_v, trows16_v, crows_v, trows_v,
                  rbuf0, rbuf1, qbuf, accbuf, sc_v, dn_v,
                  sem_s, sem0, sem1):
        wid = lax.axis_index("s") * nc + lax.axis_index("c")
        base_b = wid * bw
        lanes = lax.iota(jnp.int32, 16)

        def col(l):
            return jnp.full((16,), l, jnp.int32)

        cp_c = pltpu.make_async_copy(cidx_hbm.at[pl.ds(base_b, bw)], cidx_v, sem_s)
        cp_t = pltpu.make_async_copy(tidx_hbm.at[pl.ds(base_b, bw)], tidx_v, sem_s)
        cp_a = pltpu.make_async_copy(av_hbm.at[pl.ds(base_b * K, bw * K)], av_v, sem_s)
        cp_c.start(); cp_t.start(); cp_a.start()
        cp_c.wait(); cp_t.wait(); cp_a.wait()
        cp_cr = pltpu.make_async_copy(w16_hbm.at[cidx_v], crows16_v, sem_s)
        cp_tr = pltpu.make_async_copy(w16_hbm.at[tidx_v], trows16_v, sem_s)
        cp_cr.start(); cp_tr.start()

        rbufs = (rbuf0, rbuf1)
        sems = (sem0, sem1)

        def start_gather(lb, t, buf, sem):
            if t == 0:
                src = w16_hbm.at[av_v.at[pl.ds(lb * K, C0)]]
                pltpu.make_async_copy(src, buf, sem).start()
            else:
                src = w16_hbm.at[av_v.at[pl.ds(lb * K + C0, C1)]]
                pltpu.make_async_copy(src, buf.at[pl.ds(0, C1)], sem).start()

        def wait_gather(t, buf, sem):
            if t == 0:
                src = w16_hbm.at[av_v.at[pl.ds(0, C0)]]
                pltpu.make_async_copy(src, buf, sem).wait()
            else:
                src = w16_hbm.at[av_v.at[pl.ds(C0, C1)]]
                pltpu.make_async_copy(src, buf.at[pl.ds(0, C1)], sem).wait()

        start_gather(0, 0, rbuf0, sem0)
        start_gather(0, 1, rbuf1, sem1)
        cp_cr.wait(); cp_tr.wait()

        hi_mask = jnp.full((16,), 0xFFFF0000, jnp.uint32)

        def unpack_bf16(v32):
            w = plsc.bitcast(v32, jnp.uint32)
            even = plsc.bitcast(w << 16, jnp.float32)
            odd = plsc.bitcast(w & hi_mask, jnp.float32)
            return even, odd

        for lb in range(bw):
            for half, off in ((0, 0), (1, 32)):
                ev, od = unpack_bf16(crows16_v[lb, pl.ds(off, 32)])
                crows_v[lb, pl.ds(off, 16)] = ev
                crows_v[lb, pl.ds(off + 16, 16)] = od
                ev, od = unpack_bf16(trows16_v[lb, pl.ds(off, 32)])
                trows_v[lb, pl.ds(off, 16)] = ev
                trows_v[lb, pl.ds(off + 16, 16)] = od

        def compute_chunk(lb, t, rbuf, acc):
            ce0 = crows_v[lb, pl.ds(0, 16)]
            co0 = crows_v[lb, pl.ds(16, 16)]
            ce1 = crows_v[lb, pl.ds(32, 16)]
            co1 = crows_v[lb, pl.ds(48, 16)]

            def group(gi, acc):
                for r in range(16):
                    row = gi * 16 + r
                    e0, o0 = unpack_bf16(rbuf[row, pl.ds(0, 32)])
                    e1, o1 = unpack_bf16(rbuf[row, pl.ds(32, 32)])
                    q = e0 * ce0
                    q = q + o0 * co0
                    q = q + e1 * ce1
                    q = q + o1 * co1
                    qbuf[r] = q
                d = jnp.zeros((16,), jnp.float32)
                for l in range(16):
                    d = d + plsc.load_gather(qbuf, [lanes, col(l)])
                e = jnp.exp(d)
                if t == 1:
                    e = jnp.where(gi * 16 + lanes < C1, e, jnp.float32(0.0))
                return acc + e

            ngroups = C0 // 16 if t == 0 else (C1 + 15) // 16
            return lax.fori_loop(0, ngroups, group, acc)

        def body(i, acc):
            lb = i
            for t in range(2):
                wait_gather(t, rbufs[t], sems[t])
                acc = compute_chunk(lb, t, rbufs[t], acc)

                @pl.when(lb + 1 < bw)
                def _():
                    start_gather(lb + 1, t, rbufs[t], sems[t])
            accbuf[lb] = acc
            return jnp.zeros((16,), jnp.float32)

        lax.fori_loop(0, bw, body, jnp.zeros((16,), jnp.float32))

        for half in range(bw // 16):
            base = half * 16
            d = jnp.zeros((16,), jnp.float32)
            for l in range(16):
                d = d + plsc.load_gather(accbuf, [base + lanes, col(l)])
            dn_v[pl.ds(base, 16)] = d

        for half in range(bw // 16):
            for r in range(16):
                lb = half * 16 + r
                q = crows_v[lb, pl.ds(0, 16)] * trows_v[lb, pl.ds(0, 16)]
                q = q + crows_v[lb, pl.ds(16, 16)] * trows_v[lb, pl.ds(16, 16)]
                q = q + crows_v[lb, pl.ds(32, 16)] * trows_v[lb, pl.ds(32, 16)]
                q = q + crows_v[lb, pl.ds(48, 16)] * trows_v[lb, pl.ds(48, 16)]
                qbuf[r] = q
            d = jnp.zeros((16,), jnp.float32)
            for l in range(16):
                d = d + plsc.load_gather(qbuf, [lanes, col(l)])
            sc_v[pl.ds(half * 16, 16)] = d

        pltpu.sync_copy(sc_v, scores_hbm.at[pl.ds(base_b, bw)])
        pltpu.sync_copy(dn_v, denom_hbm.at[pl.ds(base_b, bw)])

    return sc_kernel


_sc_kernel = _sc_kernel_make()


def _finish(s_ref, d_ref, o_ref):
    nll = -jnp.mean(s_ref[...] - jnp.log(d_ref[...]))
    o_ref[...] = jnp.full((8, 128), nll, jnp.float32)


_finish_call = pl.pallas_call(
    _finish,
    out_shape=jax.ShapeDtypeStruct((8, 128), jnp.float32),
)


@jax.jit
def kernel(center_words, target_words, all_vocabs, V, U):
    cidx = center_words.reshape(-1).astype(jnp.int32)
    tidx = target_words.reshape(-1).astype(jnp.int32)
    av = all_vocabs.astype(jnp.int32).reshape(-1)
    w16 = jnp.concatenate([U, V]).astype(jnp.bfloat16)
    cidx = cidx + U.shape[0]
    scores, denom = _sc_kernel(cidx, tidx, av, w16)
    out = _finish_call(scores.reshape(8, 128), denom.reshape(8, 128))
    return out[0, 0]


# speedup vs baseline: 1.5902x; 1.1008x over previous
import functools

import jax
import jax.numpy as jnp
from jax import lax
from jax.experimental import pallas as pl
from jax.experimental.pallas import tpu as pltpu
from jax.experimental.pallas import tpu_sc as plsc

B = 1024
K = 1000
EMB = 64
C0 = 512
C1 = K - C0


def _sc_kernel_make():
    info = plsc.get_sparse_core_info()
    nc, ns = info.num_cores, info.num_subcores
    nw = nc * ns
    bw = B // nw

    mesh = plsc.VectorSubcoreMesh(core_axis_name="c", subcore_axis_name="s")

    @functools.partial(
        pl.kernel,
        mesh=mesh,
        compiler_params=pltpu.CompilerParams(
            needs_layout_passes=False, use_tc_tiling_on_sc=False),
        out_type=[
            jax.ShapeDtypeStruct((B,), jnp.float32),
            jax.ShapeDtypeStruct((B,), jnp.float32),
        ],
        scratch_types=[
            pltpu.VMEM((bw,), jnp.int32),
            pltpu.VMEM((bw,), jnp.int32),
            pltpu.VMEM((bw * K,), jnp.int32),
            pltpu.VMEM((bw, EMB), jnp.bfloat16),
            pltpu.VMEM((bw, EMB), jnp.bfloat16),
            pltpu.VMEM((bw, EMB), jnp.float32),
            pltpu.VMEM((bw, EMB), jnp.float32),
            pltpu.VMEM((C0, EMB), jnp.bfloat16),
            pltpu.VMEM((C0, EMB), jnp.bfloat16),
            pltpu.VMEM((16, 16), jnp.float32),
            pltpu.VMEM((bw, 16), jnp.float32),
            pltpu.VMEM((bw,), jnp.float32),
            pltpu.VMEM((bw,), jnp.float32),
            pltpu.SemaphoreType.DMA,
            pltpu.SemaphoreType.DMA,
            pltpu.SemaphoreType.DMA,
        ],
    )
    def sc_kernel(cidx_hbm, tidx_hbm, av_hbm, v16_hbm, u16_hbm,
                  scores_hbm, denom_hbm,
                  cidx_v, tidx_v, av_v, crows16_v, trows16_v, crows_v, trows_v,
                  rbuf0, rbuf1, qbuf, accbuf, sc_v, dn_v,
                  sem_s, sem0, sem1):
        wid = lax.axis_index("s") * nc + lax.axis_index("c")
        base_b = wid * bw
        lanes = lax.iota(jnp.int32, 16)

        def col(l):
            return jnp.full((16,), l, jnp.int32)

        cp_c = pltpu.make_async_copy(cidx_hbm.at[pl.ds(base_b, bw)], cidx_v, sem_s)
        cp_t = pltpu.make_async_copy(tidx_hbm.at[pl.ds(base_b, bw)], tidx_v, sem_s)
        cp_a = pltpu.make_async_copy(av_hbm.at[pl.ds(base_b * K, bw * K)], av_v, sem_s)
        cp_c.start(); cp_t.start(); cp_a.start()
        cp_c.wait(); cp_t.wait(); cp_a.wait()
        cp_cr = pltpu.make_async_copy(v16_hbm.at[cidx_v], crows16_v, sem_s)
        cp_tr = pltpu.make_async_copy(u16_hbm.at[tidx_v], trows16_v, sem_s)
        cp_cr.start(); cp_tr.start()

        rbufs = (rbuf0, rbuf1)
        sems = (sem0, sem1)

        def start_gather(lb, t, buf, sem):
            if t == 0:
                src = u16_hbm.at[av_v.at[pl.ds(lb * K, C0)]]
                pltpu.make_async_copy(src, buf, sem).start()
            else:
                src = u16_hbm.at[av_v.at[pl.ds(lb * K + C0, C1)]]
                pltpu.make_async_copy(src, buf.at[pl.ds(0, C1)], sem).start()

        def wait_gather(t, buf, sem):
            if t == 0:
                src = u16_hbm.at[av_v.at[pl.ds(0, C0)]]
                pltpu.make_async_copy(src, buf, sem).wait()
            else:
                src = u16_hbm.at[av_v.at[pl.ds(C0, C1)]]
                pltpu.make_async_copy(src, buf.at[pl.ds(0, C1)], sem).wait()

        start_gather(0, 0, rbuf0, sem0)
        start_gather(0, 1, rbuf1, sem1)
        cp_cr.wait(); cp_tr.wait()

        hi_mask = jnp.full((16,), 0xFFFF0000, jnp.uint32)

        def unpack_bf16(v32):
            w = plsc.bitcast(v32, jnp.uint32)
            even = plsc.bitcast(w << 16, jnp.float32)
            odd = plsc.bitcast(w & hi_mask, jnp.float32)
            return even, odd

        for lb in range(bw):
            for half, off in ((0, 0), (1, 32)):
                ev, od = unpack_bf16(crows16_v[lb, pl.ds(off, 32)])
                crows_v[lb, pl.ds(off, 16)] = ev
                crows_v[lb, pl.ds(off + 16, 16)] = od
                ev, od = unpack_bf16(trows16_v[lb, pl.ds(off, 32)])
                trows_v[lb, pl.ds(off, 16)] = ev
                trows_v[lb, pl.ds(off + 16, 16)] = od

        def compute_chunk(lb, t, rbuf, acc):
            ce0 = crows_v[lb, pl.ds(0, 16)]
            co0 = crows_v[lb, pl.ds(16, 16)]
            ce1 = crows_v[lb, pl.ds(32, 16)]
            co1 = crows_v[lb, pl.ds(48, 16)]

            def group(gi, acc):
                for r in range(16):
                    row = gi * 16 + r
                    e0, o0 = unpack_bf16(rbuf[row, pl.ds(0, 32)])
                    e1, o1 = unpack_bf16(rbuf[row, pl.ds(32, 32)])
                    q = e0 * ce0
                    q = q + o0 * co0
                    q = q + e1 * ce1
                    q = q + o1 * co1
                    qbuf[r] = q
                d = jnp.zeros((16,), jnp.float32)
                for l in range(16):
                    d = d + plsc.load_gather(qbuf, [lanes, col(l)])
                e = jnp.exp(d)
                if t == 1:
                    e = jnp.where(gi * 16 + lanes < C1, e, jnp.float32(0.0))
                return acc + e

            ngroups = C0 // 16 if t == 0 else (C1 + 15) // 16
            return lax.fori_loop(0, ngroups, group, acc)

        def body(i, acc):
            lb = i
            for t in range(2):
                wait_gather(t, rbufs[t], sems[t])
                acc = compute_chunk(lb, t, rbufs[t], acc)

                @pl.when(lb + 1 < bw)
                def _():
                    start_gather(lb + 1, t, rbufs[t], sems[t])
            accbuf[lb] = acc
            return jnp.zeros((16,), jnp.float32)

        lax.fori_loop(0, bw, body, jnp.zeros((16,), jnp.float32))

        for half in range(bw // 16):
            base = half * 16
            d = jnp.zeros((16,), jnp.float32)
            for l in range(16):
                d = d + plsc.load_gather(accbuf, [base + lanes, col(l)])
            dn_v[pl.ds(base, 16)] = d

        for half in range(bw // 16):
            for r in range(16):
                lb = half * 16 + r
                q = crows_v[lb, pl.ds(0, 16)] * trows_v[lb, pl.ds(0, 16)]
                q = q + crows_v[lb, pl.ds(16, 16)] * trows_v[lb, pl.ds(16, 16)]
                q = q + crows_v[lb, pl.ds(32, 16)] * trows_v[lb, pl.ds(32, 16)]
                q = q + crows_v[lb, pl.ds(48, 16)] * trows_v[lb, pl.ds(48, 16)]
                qbuf[r] = q
            d = jnp.zeros((16,), jnp.float32)
            for l in range(16):
                d = d + plsc.load_gather(qbuf, [lanes, col(l)])
            sc_v[pl.ds(half * 16, 16)] = d

        pltpu.sync_copy(sc_v, scores_hbm.at[pl.ds(base_b, bw)])
        pltpu.sync_copy(dn_v, denom_hbm.at[pl.ds(base_b, bw)])

    return sc_kernel


_sc_kernel = _sc_kernel_make()


def _finish(s_ref, d_ref, o_ref):
    nll = -jnp.mean(s_ref[...] - jnp.log(d_ref[...]))
    o_ref[...] = jnp.full((8, 128), nll, jnp.float32)


_finish_call = pl.pallas_call(
    _finish,
    out_shape=jax.ShapeDtypeStruct((8, 128), jnp.float32),
)


@jax.jit
def kernel(center_words, target_words, all_vocabs, V, U):
    cidx = center_words.reshape(-1).astype(jnp.int32)
    tidx = target_words.reshape(-1).astype(jnp.int32)
    av = all_vocabs.astype(jnp.int32).reshape(-1)
    scores, denom = _sc_kernel(cidx, tidx, av,
                               V.astype(jnp.bfloat16), U.astype(jnp.bfloat16))
    out = _finish_call(scores.reshape(8, 128), denom.reshape(8, 128))
    return out[0, 0]
